# baseline (speedup 1.0000x reference)
import functools

import jax
import jax.numpy as jnp
from jax import lax
from jax.experimental import pallas as pl
from jax.experimental.pallas import tpu as pltpu
from jax.experimental.pallas import tpu_sc as plsc

NN = 10000
EE = 320000
D = 16
DEE = 64
NF = 256
NG = 64
EPS = 1e-5

NC = 2
NS = 16
NW = NC * NS
EW = EE // NW
CH = 125
NCH = EW // CH
ZR = NN // NS
RB = 8

BE = 8000


def _sc_mesh():
    return plsc.VectorSubcoreMesh(
        core_axis_name="c", subcore_axis_name="s", num_cores=NC, num_subcores=NS)


def _sc_gather(x, idx_r):
    nch = idx_r.shape[1]

    @functools.partial(
        pl.kernel,
        out_type=jax.ShapeDtypeStruct((NW, nch, CH, D), jnp.float32),
        mesh=_sc_mesh(),
        scratch_types=[
            pltpu.VMEM((nch, CH), jnp.int32),
            pltpu.VMEM((RB, CH, D), jnp.float32),
            pltpu.SemaphoreType.DMA,
            pltpu.SemaphoreType.DMA,
        ],
        compiler_params=pltpu.CompilerParams(use_tc_tiling_on_sc=False),
    )
    def gk(x_hbm, idx_hbm, out_hbm, idx_v, rows_v, gsem, osem):
        c = lax.axis_index("c")
        s = lax.axis_index("s")
        w = c * NS + s
        pltpu.sync_copy(idx_hbm.at[w], idx_v)

        def body(g, carry):
            for b in range(RB):
                j = g * RB + b
                pltpu.async_copy(x_hbm.at[idx_v.at[j]], rows_v.at[b], gsem)
            for b in range(RB):
                j = g * RB + b
                pltpu.make_async_copy(
                    x_hbm.at[idx_v.at[j]], rows_v.at[b], gsem).wait()
                pltpu.async_copy(rows_v.at[b], out_hbm.at[w, j], osem)
            for b in range(RB):
                j = g * RB + b
                pltpu.make_async_copy(
                    rows_v.at[b], out_hbm.at[w, j], osem).wait()
            return carry

        lax.fori_loop(0, nch // RB, body, 0)

    return gk(x, idx_r)


def _sc_scatter_add(msg_r, idx_r, zrows):
    nch = idx_r.shape[1]

    @functools.partial(
        pl.kernel,
        out_type=jax.ShapeDtypeStruct((NC, NN, D), jnp.float32),
        mesh=_sc_mesh(),
        scratch_types=[
            pltpu.VMEM((nch, CH), jnp.int32),
            pltpu.VMEM((RB, CH, D), jnp.float32),
            pltpu.VMEM_SHARED((NN, D), jnp.float32),
            pltpu.SemaphoreType.DMA,
            pltpu.SemaphoreType.DMA,
        ],
        compiler_params=pltpu.CompilerParams(use_tc_tiling_on_sc=False),
    )
    def sk(msg_hbm, idx_hbm, z_hbm, out_hbm, idx_v, rows_v, acc_sh, lsem, ssem):
        c = lax.axis_index("c")
        s = lax.axis_index("s")
        w = c * NS + s
        pltpu.sync_copy(z_hbm, acc_sh.at[pl.ds(s * ZR, ZR)])
        pltpu.sync_copy(idx_hbm.at[w], idx_v)
        plsc.subcore_barrier()

        def body(g, carry):
            for b in range(RB):
                j = g * RB + b
                pltpu.async_copy(msg_hbm.at[w, j], rows_v.at[b], lsem)
            for b in range(RB):
                j = g * RB + b
                pltpu.make_async_copy(
                    msg_hbm.at[w, j], rows_v.at[b], lsem).wait()
                pltpu.async_copy(rows_v.at[b], acc_sh.at[idx_v.at[j]], ssem,
                                 add=True)
            for b in range(RB):
                j = g * RB + b
                pltpu.make_async_copy(
                    rows_v.at[b], acc_sh.at[idx_v.at[j]], ssem).wait()
            return carry

        lax.fori_loop(0, nch // RB, body, 0)
        plsc.subcore_barrier()
        pltpu.sync_copy(acc_sh.at[pl.ds(s * ZR, ZR)],
                        out_hbm.at[c, pl.ds(s * ZR, ZR)])

    return sk(msg_r, idx_r, zrows)


def _msg_body(eap_ref, xjp_ref, w1bd_ref, b1p_ref, w2bd_ref, b2p_ref,
              rp_ref, sp_ref, out_ref):
    h = jnp.maximum(
        jnp.dot(eap_ref[...], w1bd_ref[...], preferred_element_type=jnp.float32)
        + b1p_ref[...], 0.0)
    w = jnp.dot(h, w2bd_ref[...], preferred_element_type=jnp.float32) \
        + b2p_ref[...]
    xe = jnp.dot(xjp_ref[...], rp_ref[...],
                 preferred_element_type=jnp.float32)
    out_ref[...] = jnp.dot(xe * w, sp_ref[...],
                           preferred_element_type=jnp.float32)


def _tc_messages(eap, xjp, W1bd, b1p, W2bd, b2p, Rp, Sp):
    R = BE // 8
    ne = eap.shape[0] * 8
    return pl.pallas_call(
        _msg_body,
        grid=(ne // BE,),
        in_specs=[
            pl.BlockSpec((R, 128), lambda i: (i, 0)),
            pl.BlockSpec((R, 128), lambda i: (i, 0)),
            pl.BlockSpec((128, 8 * DEE), lambda i: (0, 0)),
            pl.BlockSpec((1, 8 * DEE), lambda i: (0, 0)),
            pl.BlockSpec((8 * DEE, 8 * D * D), lambda i: (0, 0)),
            pl.BlockSpec((1, 8 * D * D), lambda i: (0, 0)),
            pl.BlockSpec((128, 8 * D * D), lambda i: (0, 0)),
            pl.BlockSpec((8 * D * D, 128), lambda i: (0, 0)),
        ],
        out_specs=pl.BlockSpec((R, 128), lambda i: (i, 0)),
        out_shape=jax.ShapeDtypeStruct((ne // 8, 128), jnp.float32),
    )(eap, xjp, W1bd, b1p, W2bd, b2p, Rp, Sp)


def _bn_relu_of(parts):
    a = parts[0]
    for i in range(1, parts.shape[0]):
        a = a + parts[i]
    r = jnp.maximum(a, 0.0)
    ones_row = jnp.ones((1, NN), jnp.float32)
    mu = jnp.dot(ones_row, r, preferred_element_type=jnp.float32) / NN
    m2 = jnp.dot(ones_row, r * r, preferred_element_type=jnp.float32) / NN
    var = m2 - mu * mu
    return r, mu, var


def _bn_body(acc_ref, g_ref, b_ref, out_ref):
    r, mu, var = _bn_relu_of(acc_ref)
    out_ref[...] = (r - mu) * lax.rsqrt(var + EPS) * g_ref[...] + b_ref[...]


def _tc_bn_relu(parts, gamma, beta):
    return pl.pallas_call(
        _bn_body,
        out_shape=jax.ShapeDtypeStruct((NN, D), jnp.float32),
    )(parts, gamma.reshape(1, D), beta.reshape(1, D))


def _final_body(acc_ref, g_ref, b_ref, fb_ref, fbT_ref, gbT_ref,
                out_f_ref, out_g_ref):
    r, mu, var = _bn_relu_of(acc_ref)
    x2 = (r - mu) * lax.rsqrt(var + EPS) * g_ref[...] + b_ref[...]
    fb = fb_ref[...]
    fbT = fbT_ref[...]
    gbT = gbT_ref[...]
    ind_f = (fb == lax.broadcasted_iota(jnp.int32, (1, NF), 1)
             ).astype(jnp.float32)
    ind_fT = (fbT == lax.broadcasted_iota(jnp.int32, (NF, 1), 0)
              ).astype(jnp.float32)
    ind_gT = (gbT == lax.broadcasted_iota(jnp.int32, (NG, 1), 0)
              ).astype(jnp.float32)
    ones_col = jnp.ones((NN, 1), jnp.float32)
    counts = jnp.dot(ind_fT, ones_col, preferred_element_type=jnp.float32)
    npg = jnp.dot(ind_f, counts, preferred_element_type=jnp.float32)
    xn = x2 / npg
    xn_hi = xn.astype(jnp.bfloat16).astype(jnp.float32)
    xn_lo = xn - xn_hi
    out_f_ref[...] = (jnp.dot(ind_fT, xn_hi, preferred_element_type=jnp.float32)
                      + jnp.dot(ind_fT, xn_lo, preferred_element_type=jnp.float32))
    out_g_ref[...] = (jnp.dot(ind_gT, xn_hi, preferred_element_type=jnp.float32)
                      + jnp.dot(ind_gT, xn_lo, preferred_element_type=jnp.float32))


def _tc_final(parts, gamma, beta, fb, fbT, gbT):
    return pl.pallas_call(
        _final_body,
        out_shape=(jax.ShapeDtypeStruct((NF, D), jnp.float32),
                   jax.ShapeDtypeStruct((NG, D), jnp.float32)),
    )(parts, gamma.reshape(1, D), beta.reshape(1, D), fb, fbT, gbT)


def _pack_weights(W1, b1, W2, b2):
    eye8 = jnp.eye(8, dtype=jnp.float32)
    rsel = (jnp.arange(D * D, dtype=jnp.int32)[None, :] // D
            == jnp.arange(D, dtype=jnp.int32)[:, None]).astype(jnp.float32)
    ssel = (jnp.arange(D * D, dtype=jnp.int32)[:, None] % D
            == jnp.arange(D, dtype=jnp.int32)[None, :]).astype(jnp.float32)
    W1bd = jnp.kron(eye8, W1)
    W2bd = jnp.kron(eye8, W2)
    Rp = jnp.kron(eye8, rsel)
    Sp = jnp.kron(eye8, ssel)
    b1p = jnp.tile(b1, 8).reshape(1, 8 * DEE)
    b2p = jnp.tile(b2, 8).reshape(1, 8 * D * D)
    return W1bd, b1p, W2bd, b2p, Rp, Sp


def kernel(x, edge_index, edge_attr, frag_batch, graph_batch,
           W1_0, b1_0, W2_0, b2_0, gamma_0, beta_0,
           W1_1, b1_1, W2_1, b2_1, gamma_1, beta_1):
    EH = EE // 2
    NCH2 = NCH // 2
    src = edge_index[0]
    dst = edge_index[1]
    src_h = (src[:EH].reshape(NW, NCH2, CH), src[EH:].reshape(NW, NCH2, CH))
    dst_h = (dst[:EH].reshape(NW, NCH2, CH), dst[EH:].reshape(NW, NCH2, CH))
    zrows = jnp.zeros((ZR, D), jnp.float32)
    eap = edge_attr.reshape(EE // 8, 128)
    eap_h = (eap[:EH // 8], eap[EH // 8:])
    pks = (_pack_weights(W1_0, b1_0, W2_0, b2_0),
           _pack_weights(W1_1, b1_1, W2_1, b2_1))

    def layer(xin, li):
        accs = []
        xjs = [None, None]
        msgs = [None, None]
        for hh in (0, 1):
            xjs[hh] = _sc_gather(xin, src_h[hh]).reshape(EH // 8, 128)
            msgs[hh] = _tc_messages(eap_h[hh], xjs[hh], *pks[li])
        for hh in (0, 1):
            accs.append(_sc_scatter_add(
                msgs[hh].reshape(NW, NCH2, CH, D), dst_h[hh], zrows))
        return jnp.concatenate(accs, axis=0)

    acc0 = layer(x, 0)
    x1 = _tc_bn_relu(acc0, gamma_0, beta_0)
    acc1 = layer(x1, 1)

    fb = frag_batch.reshape(NN, 1)
    fbT = frag_batch.reshape(1, NN)
    gbT = graph_batch.reshape(1, NN)
    return _tc_final(acc1, gamma_1, beta_1, fb, fbT, gbT)


# speedup vs baseline: 1.2565x; 1.2565x over previous
import functools

import jax
import jax.numpy as jnp
from jax import lax
from jax.experimental import pallas as pl
from jax.experimental.pallas import tpu as pltpu
from jax.experimental.pallas import tpu_sc as plsc

NN = 10000
EE = 320000
D = 16
DEE = 64
NF = 256
NG = 64
EPS = 1e-5

NC = 2
NS = 16
NW = NC * NS
EW = EE // NW
CH = 125
NCH = EW // CH
ZR = NN // NS
RB = 8

BE = 8000


def _sc_mesh():
    return plsc.VectorSubcoreMesh(
        core_axis_name="c", subcore_axis_name="s", num_cores=NC, num_subcores=NS)


def _sc_gather(x, idx_r):
    nch = idx_r.shape[1]

    @functools.partial(
        pl.kernel,
        out_type=jax.ShapeDtypeStruct((NW, nch, CH, D), jnp.float32),
        mesh=_sc_mesh(),
        scratch_types=[
            pltpu.VMEM((nch, CH), jnp.int32),
            pltpu.VMEM((RB, CH, D), jnp.float32),
            pltpu.SemaphoreType.DMA,
            pltpu.SemaphoreType.DMA,
        ],
        compiler_params=pltpu.CompilerParams(use_tc_tiling_on_sc=False),
    )
    def gk(x_hbm, idx_hbm, out_hbm, idx_v, rows_v, gsem, osem):
        c = lax.axis_index("c")
        s = lax.axis_index("s")
        w = c * NS + s
        pltpu.sync_copy(idx_hbm.at[w], idx_v)

        def body(g, carry):
            for b in range(RB):
                j = g * RB + b
                pltpu.async_copy(x_hbm.at[idx_v.at[j]], rows_v.at[b], gsem)
            for b in range(RB):
                j = g * RB + b
                pltpu.make_async_copy(
                    x_hbm.at[idx_v.at[j]], rows_v.at[b], gsem).wait()
                pltpu.async_copy(rows_v.at[b], out_hbm.at[w, j], osem)
            for b in range(RB):
                j = g * RB + b
                pltpu.make_async_copy(
                    rows_v.at[b], out_hbm.at[w, j], osem).wait()
            return carry

        lax.fori_loop(0, nch // RB, body, 0)

    return gk(x, idx_r)


def _sc_scatter_add(msg_r, idx_r, zrows):
    nch = idx_r.shape[1]

    @functools.partial(
        pl.kernel,
        out_type=jax.ShapeDtypeStruct((NC, NN, D), jnp.float32),
        mesh=_sc_mesh(),
        scratch_types=[
            pltpu.VMEM((nch, CH), jnp.int32),
            pltpu.VMEM((RB, CH, D), jnp.float32),
            pltpu.VMEM_SHARED((NN, D), jnp.float32),
            pltpu.SemaphoreType.DMA,
            pltpu.SemaphoreType.DMA,
        ],
        compiler_params=pltpu.CompilerParams(use_tc_tiling_on_sc=False),
    )
    def sk(msg_hbm, idx_hbm, z_hbm, out_hbm, idx_v, rows_v, acc_sh, lsem, ssem):
        c = lax.axis_index("c")
        s = lax.axis_index("s")
        w = c * NS + s
        pltpu.sync_copy(z_hbm, acc_sh.at[pl.ds(s * ZR, ZR)])
        pltpu.sync_copy(idx_hbm.at[w], idx_v)
        plsc.subcore_barrier()

        def body(g, carry):
            for b in range(RB):
                j = g * RB + b
                pltpu.async_copy(msg_hbm.at[w, j], rows_v.at[b], lsem)
            for b in range(RB):
                j = g * RB + b
                pltpu.make_async_copy(
                    msg_hbm.at[w, j], rows_v.at[b], lsem).wait()
                pltpu.async_copy(rows_v.at[b], acc_sh.at[idx_v.at[j]], ssem,
                                 add=True)
            for b in range(RB):
                j = g * RB + b
                pltpu.make_async_copy(
                    rows_v.at[b], acc_sh.at[idx_v.at[j]], ssem).wait()
            return carry

        lax.fori_loop(0, nch // RB, body, 0)
        plsc.subcore_barrier()
        pltpu.sync_copy(acc_sh.at[pl.ds(s * ZR, ZR)],
                        out_hbm.at[c, pl.ds(s * ZR, ZR)])

    return sk(msg_r, idx_r, zrows)


def _msg_body(eap_ref, xjp_ref, w1bd_ref, b1p_ref, w2bd_ref, b2p_ref,
              rp_ref, sp_ref, out_ref):
    h = jnp.maximum(
        jnp.dot(eap_ref[...], w1bd_ref[...], preferred_element_type=jnp.float32)
        + b1p_ref[...], 0.0)
    w = jnp.dot(h, w2bd_ref[...], preferred_element_type=jnp.float32) \
        + b2p_ref[...]
    xe = jnp.dot(xjp_ref[...].astype(jnp.bfloat16),
                 rp_ref[...].astype(jnp.bfloat16),
                 preferred_element_type=jnp.float32)
    out_ref[...] = jnp.dot(xe * w, sp_ref[...],
                           preferred_element_type=jnp.float32)


def _tc_messages(eap, xjp, W1bd, b1p, W2bd, b2p, Rp, Sp):
    R = BE // 8
    ne = eap.shape[0] * 8
    return pl.pallas_call(
        _msg_body,
        grid=(ne // BE,),
        in_specs=[
            pl.BlockSpec((R, 128), lambda i: (i, 0)),
            pl.BlockSpec((R, 128), lambda i: (i, 0)),
            pl.BlockSpec((128, 8 * DEE), lambda i: (0, 0)),
            pl.BlockSpec((1, 8 * DEE), lambda i: (0, 0)),
            pl.BlockSpec((8 * DEE, 8 * D * D), lambda i: (0, 0)),
            pl.BlockSpec((1, 8 * D * D), lambda i: (0, 0)),
            pl.BlockSpec((128, 8 * D * D), lambda i: (0, 0)),
            pl.BlockSpec((8 * D * D, 128), lambda i: (0, 0)),
        ],
        out_specs=pl.BlockSpec((R, 128), lambda i: (i, 0)),
        out_shape=jax.ShapeDtypeStruct((ne // 8, 128), jnp.float32),
    )(eap, xjp, W1bd, b1p, W2bd, b2p, Rp, Sp)


def _bn_relu_of(parts):
    a = parts[0]
    for i in range(1, parts.shape[0]):
        a = a + parts[i]
    r = jnp.maximum(a, 0.0)
    ones_row = jnp.ones((1, NN), jnp.float32)
    mu = jnp.dot(ones_row, r, preferred_element_type=jnp.float32) / NN
    m2 = jnp.dot(ones_row, r * r, preferred_element_type=jnp.float32) / NN
    var = m2 - mu * mu
    return r, mu, var


def _bn_body(acc_ref, g_ref, b_ref, out_ref):
    r, mu, var = _bn_relu_of(acc_ref)
    out_ref[...] = (r - mu) * lax.rsqrt(var + EPS) * g_ref[...] + b_ref[...]


def _tc_bn_relu(parts, gamma, beta):
    return pl.pallas_call(
        _bn_body,
        out_shape=jax.ShapeDtypeStruct((NN, D), jnp.float32),
    )(parts, gamma.reshape(1, D), beta.reshape(1, D))


def _final_body(acc_ref, g_ref, b_ref, fb_ref, fbT_ref, gbT_ref,
                out_f_ref, out_g_ref):
    r, mu, var = _bn_relu_of(acc_ref)
    x2 = (r - mu) * lax.rsqrt(var + EPS) * g_ref[...] + b_ref[...]
    fb = fb_ref[...]
    fbT = fbT_ref[...]
    gbT = gbT_ref[...]
    ind_f = (fb == lax.broadcasted_iota(jnp.int32, (1, NF), 1)
             ).astype(jnp.float32)
    ind_fT = (fbT == lax.broadcasted_iota(jnp.int32, (NF, 1), 0)
              ).astype(jnp.float32)
    ind_gT = (gbT == lax.broadcasted_iota(jnp.int32, (NG, 1), 0)
              ).astype(jnp.float32)
    ones_col = jnp.ones((NN, 1), jnp.float32)
    counts = jnp.dot(ind_fT, ones_col, preferred_element_type=jnp.float32)
    npg = jnp.dot(ind_f, counts, preferred_element_type=jnp.float32)
    xn = x2 / npg
    xn_hi = xn.astype(jnp.bfloat16).astype(jnp.float32)
    xn_lo = xn - xn_hi
    out_f_ref[...] = (jnp.dot(ind_fT, xn_hi, preferred_element_type=jnp.float32)
                      + jnp.dot(ind_fT, xn_lo, preferred_element_type=jnp.float32))
    out_g_ref[...] = (jnp.dot(ind_gT, xn_hi, preferred_element_type=jnp.float32)
                      + jnp.dot(ind_gT, xn_lo, preferred_element_type=jnp.float32))


def _tc_final(parts, gamma, beta, fb, fbT, gbT):
    return pl.pallas_call(
        _final_body,
        out_shape=(jax.ShapeDtypeStruct((NF, D), jnp.float32),
                   jax.ShapeDtypeStruct((NG, D), jnp.float32)),
    )(parts, gamma.reshape(1, D), beta.reshape(1, D), fb, fbT, gbT)


def _pack_weights(W1, b1, W2, b2):
    eye8 = jnp.eye(8, dtype=jnp.float32)
    rsel = (jnp.arange(D * D, dtype=jnp.int32)[None, :] // D
            == jnp.arange(D, dtype=jnp.int32)[:, None]).astype(jnp.float32)
    ssel = (jnp.arange(D * D, dtype=jnp.int32)[:, None] % D
            == jnp.arange(D, dtype=jnp.int32)[None, :]).astype(jnp.float32)
    W1bd = jnp.kron(eye8, W1)
    W2bd = jnp.kron(eye8, W2)
    Rp = jnp.kron(eye8, rsel)
    Sp = jnp.kron(eye8, ssel)
    b1p = jnp.tile(b1, 8).reshape(1, 8 * DEE)
    b2p = jnp.tile(b2, 8).reshape(1, 8 * D * D)
    return W1bd, b1p, W2bd, b2p, Rp, Sp


def kernel(x, edge_index, edge_attr, frag_batch, graph_batch,
           W1_0, b1_0, W2_0, b2_0, gamma_0, beta_0,
           W1_1, b1_1, W2_1, b2_1, gamma_1, beta_1):
    src_r = edge_index[0].reshape(NW, NCH, CH)
    dst_r = edge_index[1].reshape(NW, NCH, CH)
    zrows = jnp.zeros((ZR, D), jnp.float32)
    eap = edge_attr.reshape(EE // 8, 128)
    pk0 = _pack_weights(W1_0, b1_0, W2_0, b2_0)
    pk1 = _pack_weights(W1_1, b1_1, W2_1, b2_1)

    xj0 = _sc_gather(x, src_r).reshape(EE // 8, 128)
    msg0 = _tc_messages(eap, xj0, *pk0)
    acc0 = _sc_scatter_add(msg0.reshape(NW, NCH, CH, D), dst_r, zrows)
    x1 = _tc_bn_relu(acc0, gamma_0, beta_0)

    xj1 = _sc_gather(x1, src_r).reshape(EE // 8, 128)
    msg1 = _tc_messages(eap, xj1, *pk1)
    acc1 = _sc_scatter_add(msg1.reshape(NW, NCH, CH, D), dst_r, zrows)

    fb = frag_batch.reshape(NN, 1)
    fbT = frag_batch.reshape(1, NN)
    gbT = graph_batch.reshape(1, NN)
    return _tc_final(acc1, gamma_1, beta_1, fb, fbT, gbT)


# speedup vs baseline: 1.2574x; 1.0007x over previous
import functools

import jax
import jax.numpy as jnp
from jax import lax
from jax.experimental import pallas as pl
from jax.experimental.pallas import tpu as pltpu
from jax.experimental.pallas import tpu_sc as plsc

NN = 10000
EE = 320000
D = 16
DEE = 64
NF = 256
NG = 64
EPS = 1e-5

NC = 2
NS = 16
NW = NC * NS
EW = EE // NW
CH = 125
NCH = EW // CH
ZR = NN // NS
RB = 8

BE = 8000


def _sc_mesh():
    return plsc.VectorSubcoreMesh(
        core_axis_name="c", subcore_axis_name="s", num_cores=NC, num_subcores=NS)


def _sc_gather(x, idx_r):
    nch = idx_r.shape[1]

    @functools.partial(
        pl.kernel,
        out_type=jax.ShapeDtypeStruct((NW, nch, CH, D), jnp.float32),
        mesh=_sc_mesh(),
        scratch_types=[
            pltpu.VMEM((nch, CH), jnp.int32),
            pltpu.VMEM((RB, CH, D), jnp.float32),
            pltpu.SemaphoreType.DMA,
            pltpu.SemaphoreType.DMA,
        ],
        compiler_params=pltpu.CompilerParams(use_tc_tiling_on_sc=False),
    )
    def gk(x_hbm, idx_hbm, out_hbm, idx_v, rows_v, gsem, osem):
        c = lax.axis_index("c")
        s = lax.axis_index("s")
        w = c * NS + s
        pltpu.sync_copy(idx_hbm.at[w], idx_v)

        def body(g, carry):
            for b in range(RB):
                j = g * RB + b
                pltpu.async_copy(x_hbm.at[idx_v.at[j]], rows_v.at[b], gsem)
            for b in range(RB):
                j = g * RB + b
                pltpu.make_async_copy(
                    x_hbm.at[idx_v.at[j]], rows_v.at[b], gsem).wait()
                pltpu.async_copy(rows_v.at[b], out_hbm.at[w, j], osem)
            for b in range(RB):
                j = g * RB + b
                pltpu.make_async_copy(
                    rows_v.at[b], out_hbm.at[w, j], osem).wait()
            return carry

        lax.fori_loop(0, nch // RB, body, 0)

    return gk(x, idx_r)


def _sc_scatter_add(msg_r, idx_r, zrows):
    nch = idx_r.shape[1]

    @functools.partial(
        pl.kernel,
        out_type=jax.ShapeDtypeStruct((NC, NN, D), jnp.float32),
        mesh=_sc_mesh(),
        scratch_types=[
            pltpu.VMEM((nch, CH), jnp.int32),
            pltpu.VMEM((RB, CH, D), jnp.float32),
            pltpu.VMEM_SHARED((NN, D), jnp.float32),
            pltpu.SemaphoreType.DMA,
            pltpu.SemaphoreType.DMA,
        ],
        compiler_params=pltpu.CompilerParams(use_tc_tiling_on_sc=False),
    )
    def sk(msg_hbm, idx_hbm, z_hbm, out_hbm, idx_v, rows_v, acc_sh, lsem, ssem):
        c = lax.axis_index("c")
        s = lax.axis_index("s")
        w = c * NS + s
        pltpu.sync_copy(z_hbm, acc_sh.at[pl.ds(s * ZR, ZR)])
        pltpu.sync_copy(idx_hbm.at[w], idx_v)
        plsc.subcore_barrier()

        def body(g, carry):
            for b in range(RB):
                j = g * RB + b
                pltpu.async_copy(msg_hbm.at[w, j], rows_v.at[b], lsem)
            for b in range(RB):
                j = g * RB + b
                pltpu.make_async_copy(
                    msg_hbm.at[w, j], rows_v.at[b], lsem).wait()
                pltpu.async_copy(rows_v.at[b], acc_sh.at[idx_v.at[j]], ssem,
                                 add=True)
            for b in range(RB):
                j = g * RB + b
                pltpu.make_async_copy(
                    rows_v.at[b], acc_sh.at[idx_v.at[j]], ssem).wait()
            return carry

        lax.fori_loop(0, nch // RB, body, 0)
        plsc.subcore_barrier()
        pltpu.sync_copy(acc_sh.at[pl.ds(s * ZR, ZR)],
                        out_hbm.at[c, pl.ds(s * ZR, ZR)])

    return sk(msg_r, idx_r, zrows)


def _msg_body(eap_ref, xjp_ref, w1bd_ref, b1p_ref, w2bd_ref, b2p_ref,
              rp_ref, sp_ref, out_ref):
    h = jnp.maximum(
        jnp.dot(eap_ref[...], w1bd_ref[...], preferred_element_type=jnp.float32)
        + b1p_ref[...], 0.0)
    w = jnp.dot(h, w2bd_ref[...], preferred_element_type=jnp.float32) \
        + b2p_ref[...]
    xe = jnp.dot(xjp_ref[...].astype(jnp.bfloat16), rp_ref[...],
                 preferred_element_type=jnp.float32)
    out_ref[...] = jnp.dot(xe * w, sp_ref[...],
                           preferred_element_type=jnp.float32)


def _tc_messages(eap, xjp, W1bd, b1p, W2bd, b2p, Rp, Sp):
    R = BE // 8
    ne = eap.shape[0] * 8
    return pl.pallas_call(
        _msg_body,
        grid=(ne // BE,),
        in_specs=[
            pl.BlockSpec((R, 128), lambda i: (i, 0)),
            pl.BlockSpec((R, 128), lambda i: (i, 0)),
            pl.BlockSpec((128, 8 * DEE), lambda i: (0, 0)),
            pl.BlockSpec((1, 8 * DEE), lambda i: (0, 0)),
            pl.BlockSpec((8 * DEE, 8 * D * D), lambda i: (0, 0)),
            pl.BlockSpec((1, 8 * D * D), lambda i: (0, 0)),
            pl.BlockSpec((128, 8 * D * D), lambda i: (0, 0)),
            pl.BlockSpec((8 * D * D, 128), lambda i: (0, 0)),
        ],
        out_specs=pl.BlockSpec((R, 128), lambda i: (i, 0)),
        out_shape=jax.ShapeDtypeStruct((ne // 8, 128), jnp.float32),
    )(eap, xjp, W1bd, b1p, W2bd, b2p, Rp, Sp)


def _bn_relu_of(parts):
    a = parts[0]
    for i in range(1, parts.shape[0]):
        a = a + parts[i]
    r = jnp.maximum(a, 0.0)
    ones_row = jnp.ones((1, NN), jnp.float32)
    mu = jnp.dot(ones_row, r, preferred_element_type=jnp.float32) / NN
    m2 = jnp.dot(ones_row, r * r, preferred_element_type=jnp.float32) / NN
    var = m2 - mu * mu
    return r, mu, var


def _bn_body(acc_ref, g_ref, b_ref, out_ref):
    r, mu, var = _bn_relu_of(acc_ref)
    out_ref[...] = (r - mu) * lax.rsqrt(var + EPS) * g_ref[...] + b_ref[...]


def _tc_bn_relu(parts, gamma, beta):
    return pl.pallas_call(
        _bn_body,
        out_shape=jax.ShapeDtypeStruct((NN, D), jnp.float32),
    )(parts, gamma.reshape(1, D), beta.reshape(1, D))


def _final_body(acc_ref, g_ref, b_ref, fb_ref, fbT_ref, gbT_ref,
                out_f_ref, out_g_ref):
    r, mu, var = _bn_relu_of(acc_ref)
    x2 = (r - mu) * lax.rsqrt(var + EPS) * g_ref[...] + b_ref[...]
    fb = fb_ref[...]
    fbT = fbT_ref[...]
    gbT = gbT_ref[...]
    ind_f = (fb == lax.broadcasted_iota(jnp.int32, (1, NF), 1)
             ).astype(jnp.float32)
    ind_fT = (fbT == lax.broadcasted_iota(jnp.int32, (NF, 1), 0)
              ).astype(jnp.float32)
    ind_gT = (gbT == lax.broadcasted_iota(jnp.int32, (NG, 1), 0)
              ).astype(jnp.float32)
    ones_col = jnp.ones((NN, 1), jnp.float32)
    counts = jnp.dot(ind_fT, ones_col, preferred_element_type=jnp.float32)
    npg = jnp.dot(ind_f, counts, preferred_element_type=jnp.float32)
    xn = x2 / npg
    xn_hi = xn.astype(jnp.bfloat16).astype(jnp.float32)
    xn_lo = xn - xn_hi
    out_f_ref[...] = (jnp.dot(ind_fT, xn_hi, preferred_element_type=jnp.float32)
                      + jnp.dot(ind_fT, xn_lo, preferred_element_type=jnp.float32))
    out_g_ref[...] = (jnp.dot(ind_gT, xn_hi, preferred_element_type=jnp.float32)
                      + jnp.dot(ind_gT, xn_lo, preferred_element_type=jnp.float32))


def _tc_final(parts, gamma, beta, fb, fbT, gbT):
    return pl.pallas_call(
        _final_body,
        out_shape=(jax.ShapeDtypeStruct((NF, D), jnp.float32),
                   jax.ShapeDtypeStruct((NG, D), jnp.float32)),
    )(parts, gamma.reshape(1, D), beta.reshape(1, D), fb, fbT, gbT)


def _pack_weights(W1, b1, W2, b2):
    eye8 = jnp.eye(8, dtype=jnp.float32)
    rsel = (jnp.arange(D * D, dtype=jnp.int32)[None, :] // D
            == jnp.arange(D, dtype=jnp.int32)[:, None]).astype(jnp.float32)
    ssel = (jnp.arange(D * D, dtype=jnp.int32)[:, None] % D
            == jnp.arange(D, dtype=jnp.int32)[None, :]).astype(jnp.float32)
    W1bd = jnp.kron(eye8, W1)
    W2bd = jnp.kron(eye8, W2)
    Rp = jnp.kron(eye8, rsel).astype(jnp.bfloat16)
    Sp = jnp.kron(eye8, ssel)
    b1p = jnp.tile(b1, 8).reshape(1, 8 * DEE)
    b2p = jnp.tile(b2, 8).reshape(1, 8 * D * D)
    return W1bd, b1p, W2bd, b2p, Rp, Sp


def kernel(x, edge_index, edge_attr, frag_batch, graph_batch,
           W1_0, b1_0, W2_0, b2_0, gamma_0, beta_0,
           W1_1, b1_1, W2_1, b2_1, gamma_1, beta_1):
    src_r = edge_index[0].reshape(NW, NCH, CH)
    dst_r = edge_index[1].reshape(NW, NCH, CH)
    zrows = jnp.zeros((ZR, D), jnp.float32)
    eap = edge_attr.reshape(EE // 8, 128)
    pk0 = _pack_weights(W1_0, b1_0, W2_0, b2_0)
    pk1 = _pack_weights(W1_1, b1_1, W2_1, b2_1)

    xj0 = _sc_gather(x, src_r).reshape(EE // 8, 128)
    msg0 = _tc_messages(eap, xj0, *pk0)
    acc0 = _sc_scatter_add(msg0.reshape(NW, NCH, CH, D), dst_r, zrows)
    x1 = _tc_bn_relu(acc0, gamma_0, beta_0)

    xj1 = _sc_gather(x1, src_r).reshape(EE // 8, 128)
    msg1 = _tc_messages(eap, xj1, *pk1)
    acc1 = _sc_scatter_add(msg1.reshape(NW, NCH, CH, D), dst_r, zrows)

    fb = frag_batch.reshape(NN, 1)
    fbT = frag_batch.reshape(1, NN)
    gbT = graph_batch.reshape(1, NN)
    return _tc_final(acc1, gamma_1, beta_1, fb, fbT, gbT)
